# baseline (speedup 1.0000x reference)
import functools

import jax
import jax.numpy as jnp
from jax import lax
from jax.experimental import pallas as pl
from jax.experimental.pallas import tpu as pltpu
from jax.experimental.pallas import tpu_sc as plsc

H = 128
NU = 50000
NI = 50000
E = 320000
NL = 2
B = 8192
NUP = 50048
NF = 4
HF = H // NF
BN = 2000
NB = NU // BN


def _ln(x, g, b, eps=1e-5):
    mu = jnp.mean(x, axis=-1, keepdims=True)
    var = jnp.mean((x - mu) ** 2, axis=-1, keepdims=True)
    return (x - mu) * lax.rsqrt(var + eps) * g + b


def _stage_a_body(xi_ref, g_ref, b_ref, o0_ref, o1_ref):
    x = xi_ref[...]
    y = jnp.maximum(_ln(x, g_ref[...], b_ref[...]), 0.0)
    for c in range(NF):
        o0_ref[c] = x[:, c * HF:(c + 1) * HF]
        o1_ref[c] = y[:, c * HF:(c + 1) * HF]


def _stage_a(xi0, g, b):
    return pl.pallas_call(
        _stage_a_body,
        grid=(NB,),
        in_specs=[
            pl.BlockSpec((BN, H), lambda i: (i, 0)),
            pl.BlockSpec((1, H), lambda i: (0, 0)),
            pl.BlockSpec((1, H), lambda i: (0, 0)),
        ],
        out_specs=[
            pl.BlockSpec((NF, BN, HF), lambda i: (0, i, 0)),
            pl.BlockSpec((NF, BN, HF), lambda i: (0, i, 0)),
        ],
        out_shape=[
            jax.ShapeDtypeStruct((NF, NUP, HF), jnp.float32),
            jax.ShapeDtypeStruct((NF, NUP, HF), jnp.float32),
        ],
    )(xi0, g.reshape(1, H), b.reshape(1, H))


EB = 128
EROWS = 2560
NBR = EROWS // 16
RB2 = 320


def _stage_a2_body(g_ref, h_ref, o_ref, p_ref):
    c = pl.program_id(0)
    o_ref[0] = g_ref[...] + c * NUP
    p_ref[0] = h_ref[...] + c * NUP


def _stage_a2(g2d, h2d):
    return pl.pallas_call(
        _stage_a2_body,
        grid=(NF, EROWS // RB2),
        in_specs=[pl.BlockSpec((RB2, EB), lambda c, r: (r, 0)),
                  pl.BlockSpec((RB2, EB), lambda c, r: (r, 0))],
        out_specs=[pl.BlockSpec((1, RB2, EB), lambda c, r: (c, r, 0)),
                   pl.BlockSpec((1, RB2, EB), lambda c, r: (c, r, 0))],
        out_shape=[jax.ShapeDtypeStruct((NF, EROWS, EB), jnp.int32),
                   jax.ShapeDtypeStruct((NF, EROWS, EB), jnp.int32)],
    )(g2d, h2d)


NSLOT = 5
RSTAGE = 40
NWIN = NBR // RSTAGE
NRND = RSTAGE // NSLOT
RPT = NUP // 16
ZCH = 128


def _sc_body(do_counts, *refs):
    if do_counts:
        (tblr, gfk4, grv4, sfk, srv,
         ma, mb, c1o, c2o,
         gst, sst, rows, acc, gsem, ssem, zsem) = refs
    else:
        (tblr, gfk4, grv4, sfk, srv,
         ma, mb,
         gst, sst, rows, acc, gsem, ssem, zsem) = refs
    core = lax.axis_index("c")
    sub = lax.axis_index("s")
    row0 = sub * RPT
    brow0 = sub * NBR

    def fill_slot0(val):
        def body(i, _):
            for k in range(HF // 16):
                rows[0, i, pl.ds(k * 16, 16)] = jnp.full((16,), val,
                                                         jnp.float32)
            return 0
        lax.fori_loop(0, EB, body, 0)

    def zero_acc():
        fill_slot0(0.0)
        zsrc = rows.at[0]
        copies = [pltpu.async_copy(
            zsrc.at[pl.ds(0, ZCH)] if z < 24 else zsrc.at[pl.ds(0, 56)],
            acc.at[pl.ds(row0 + z * ZCH, ZCH if z < 24 else 56)],
            zsem) for z in range(25)]
        for c in copies:
            c.wait()

    def edge_pass(tbl, g4, s2d, chunk):
        def window(w, _):
            wrow = brow0 + w * RSTAGE
            pltpu.sync_copy(s2d.at[pl.ds(wrow, RSTAGE)], sst)
            if tbl is not None:
                pltpu.sync_copy(g4.at[chunk, pl.ds(wrow, RSTAGE)], gst)

                def g_start(k, lr):
                    pltpu.async_copy(tbl.at[gst.at[lr]], rows.at[k],
                                     gsem.at[k])

                def g_wait(k, lr):
                    pltpu.make_async_copy(tbl.at[gst.at[lr]], rows.at[k],
                                          gsem.at[k]).wait()

                def s_start(k, lr):
                    pltpu.async_copy(rows.at[k], acc.at[sst.at[lr]],
                                     ssem.at[k], add=True)

                def s_wait(k, lr):
                    pltpu.make_async_copy(rows.at[k], acc.at[sst.at[lr]],
                                          ssem.at[k]).wait()

                for k in range(NSLOT):
                    g_start(k, k)

                def rnd(r, _):
                    for k in range(NSLOT):
                        lr = r * NSLOT + k
                        g_wait(k, lr)
                        s_start(k, lr)
                    for k in range(NSLOT):
                        lr = r * NSLOT + k
                        s_wait(k, lr)

                        @pl.when(r < NRND - 1)
                        def _():
                            g_start(k, lr + NSLOT)
                    return 0
                lax.fori_loop(0, NRND, rnd, 0)
            else:
                def rnd(r, _):
                    copies = [pltpu.async_copy(
                        rows.at[0], acc.at[sst.at[r * NSLOT + k]],
                        ssem.at[k], add=True) for k in range(NSLOT)]
                    for c in copies:
                        c.wait()
                    return 0
                lax.fori_loop(0, NRND, rnd, 0)
            return 0
        lax.fori_loop(0, NWIN, window, 0)

    def writeback(out, toff):
        pltpu.sync_copy(acc.at[pl.ds(row0, RPT)],
                        out.at[pl.ds(toff + row0, RPT)])

    if do_counts:
        zero_acc()
        fill_slot0(1.0)
        plsc.subcore_barrier()

        @pl.when(core == 0)
        def _():
            edge_pass(None, None, sfk, 0)

        @pl.when(core == 1)
        def _():
            edge_pass(None, None, srv, 0)
        plsc.subcore_barrier()

        @pl.when(core == 0)
        def _():
            writeback(c1o, 0)

        @pl.when(core == 1)
        def _():
            writeback(c2o, 0)
        plsc.subcore_barrier()

    for g4, s2d, out in ((gfk4, sfk, ma), (grv4, srv, mb)):
        def chunk_body(j, _, g4=g4, s2d=s2d, out=out):
            chunk = core * 2 + j
            zero_acc()
            plsc.subcore_barrier()
            edge_pass(tblr, g4, s2d, chunk)
            plsc.subcore_barrier()
            writeback(out, chunk * NUP)
            plsc.subcore_barrier()
            return 0
        lax.fori_loop(0, 2, chunk_body, 0)


def _seg_means_sc(tbl, gfk4, grv4, sfk, srv, do_counts):
    m_ty = jax.ShapeDtypeStruct((NF * NUP, HF), jnp.float32)
    c_ty = jax.ShapeDtypeStruct((NUP, HF), jnp.float32)
    f = pl.kernel(
        functools.partial(_sc_body, do_counts),
        out_type=[m_ty, m_ty] + ([c_ty, c_ty] if do_counts else []),
        mesh=plsc.VectorSubcoreMesh(core_axis_name="c", subcore_axis_name="s"),
        compiler_params=pltpu.CompilerParams(use_tc_tiling_on_sc=False),
        scratch_types=[
            pltpu.VMEM((RSTAGE, EB), jnp.int32),
            pltpu.VMEM((RSTAGE, EB), jnp.int32),
            pltpu.VMEM((NSLOT, EB, HF), jnp.float32),
            pltpu.VMEM_SHARED((NUP, HF), jnp.float32),
            pltpu.SemaphoreType.DMA((NSLOT,)),
            pltpu.SemaphoreType.DMA((NSLOT,)),
            pltpu.SemaphoreType.DMA,
        ],
    )
    outs = f(tbl.reshape(NF * NUP, HF), gfk4, grv4, sfk, srv)
    rs = lambda m: m.reshape(NF, NUP, HF)
    if do_counts:
        ma, mb, c1, c2 = outs
        return rs(ma), rs(mb), c1, c2
    ma, mb = outs
    return rs(ma), rs(mb)


def _pad2d(x, fill):
    x = x.reshape(E // EB, EB)
    pad = jnp.full((EROWS - E // EB, EB), fill, jnp.int32)
    return jnp.concatenate([x, pad], axis=0)


def _layer_body(xu_ref, m1_ref, m2_ref, c1_ref, c2_ref,
                w1_ref, b1_ref, w2_ref, b2_ref, wu_ref, bu_ref,
                pg_ref, pb_ref, wl_ref, bl_ref, wr_ref, ng_ref, nb_ref,
                o_ref):
    xu = xu_ref[...]
    c1 = c1_ref[:, 0:1]
    c2 = c2_ref[:, 0:1]
    r1 = 1.0 / jnp.maximum(c1, 1.0)
    r2 = 1.0 / jnp.maximum(c2, 1.0)
    has1 = (c1 > 0.0).astype(jnp.float32)

    z = jnp.maximum(
        jnp.dot(xu, w1_ref[...], preferred_element_type=jnp.float32)
        + b1_ref[...], 0.0)
    z = jnp.dot(z, w2_ref[...], preferred_element_type=jnp.float32) + b2_ref[...]

    m1 = jnp.concatenate([m1_ref[c] for c in range(NF)], axis=1)
    m2 = jnp.concatenate([m2_ref[c] for c in range(NF)], axis=1)
    agg = m1 * r1 - z * has1
    upd = jnp.dot(agg, wu_ref[...],
                  preferred_element_type=jnp.float32) + bu_ref[...]
    sage = (jnp.dot(m2 * r2, wl_ref[...], preferred_element_type=jnp.float32)
            + bl_ref[...]
            + jnp.dot(xu, wr_ref[...], preferred_element_type=jnp.float32))

    prmp = _ln(xu + upd, pg_ref[...], pb_ref[...])
    merged = (prmp + sage) * 0.5
    o_ref[...] = jnp.maximum(_ln(merged, ng_ref[...], nb_ref[...]), 0.0)


def _layer_dense(xu, m1, m2, cnt1, cnt2, p):
    vec = lambda v: v.reshape(1, H)
    full = lambda shp: pl.BlockSpec(shp, lambda i: (0, 0))
    return pl.pallas_call(
        _layer_body,
        grid=(NB,),
        in_specs=[
            pl.BlockSpec((BN, H), lambda i: (i, 0)),
            pl.BlockSpec((NF, BN, HF), lambda i: (0, i, 0)),
            pl.BlockSpec((NF, BN, HF), lambda i: (0, i, 0)),
            pl.BlockSpec((BN, HF), lambda i: (i, 0)),
            pl.BlockSpec((BN, HF), lambda i: (i, 0)),
            full((H, H)), full((1, H)), full((H, H)), full((1, H)),
            full((H, H)), full((1, H)), full((1, H)), full((1, H)),
            full((H, H)), full((1, H)), full((H, H)), full((1, H)),
            full((1, H)),
        ],
        out_specs=pl.BlockSpec((BN, H), lambda i: (i, 0)),
        out_shape=jax.ShapeDtypeStruct((NU, H), jnp.float32),
    )(xu, m1, m2, cnt1, cnt2,
      p['W1'], vec(p['b1']), p['W2'], vec(p['b2']),
      p['Wu'], vec(p['bu']), vec(p['pg']), vec(p['pb']),
      p['Wl'], vec(p['bl']), p['Wr'], vec(p['nug']), vec(p['nub']))


def _head_body(h_ref, w1_ref, b1_ref, w2_ref, b2_ref, o_ref):
    h = jnp.maximum(
        jnp.dot(h_ref[...], w1_ref[...], preferred_element_type=jnp.float32)
        + b1_ref[...], 0.0)
    o_ref[...] = (jnp.dot(h, w2_ref[...], preferred_element_type=jnp.float32)
                  + b2_ref[...])


def _head(hrows, hW1, hb1, hW2, hb2):
    BH = 1024
    out = pl.pallas_call(
        _head_body,
        grid=(B // BH,),
        in_specs=[
            pl.BlockSpec((BH, H), lambda i: (i, 0)),
            pl.BlockSpec((H, H // 2), lambda i: (0, 0)),
            pl.BlockSpec((1, H // 2), lambda i: (0, 0)),
            pl.BlockSpec((H // 2, 1), lambda i: (0, 0)),
            pl.BlockSpec((1, 1), lambda i: (0, 0)),
        ],
        out_specs=pl.BlockSpec((BH, 1), lambda i: (i, 0)),
        out_shape=jax.ShapeDtypeStruct((B, 1), jnp.float32),
    )(hrows, hW1, hb1.reshape(1, H // 2), hW2, hb2.reshape(1, 1))
    return out.reshape(B)


def kernel(params, edge_fk, edge_rev, target_ids):
    p0 = params['layer0']
    xi0 = params['emb_item']
    xu = params['emb_user']

    xi0c, xi1c = _stage_a(xi0, p0['nig'], p0['nib'])
    gfk4, grv4 = _stage_a2(_pad2d(edge_fk[1], 0), _pad2d(edge_rev[0], 0))
    sfk = _pad2d(edge_fk[0], NU)
    srv = _pad2d(edge_rev[1], NU)
    m10, m20, c1, c2 = _seg_means_sc(xi0c, gfk4, grv4, sfk, srv, True)
    m11, m21 = _seg_means_sc(xi1c, gfk4, grv4, sfk, srv, False)

    xu = _layer_dense(xu, m10, m20, c1, c2, params['layer0'])
    xu = _layer_dense(xu, m11, m21, c1, c2, params['layer1'])

    hrows = xu[target_ids]
    return _head(hrows, params['hW1'], params['hb1'], params['hW2'], params['hb2'])


# speedup vs baseline: 1.0079x; 1.0079x over previous
import functools

import jax
import jax.numpy as jnp
from jax import lax
from jax.experimental import pallas as pl
from jax.experimental.pallas import tpu as pltpu
from jax.experimental.pallas import tpu_sc as plsc

H = 128
NU = 50000
NI = 50000
E = 320000
NL = 2
B = 8192
NUP = 50048
NF = 4
HF = H // NF
BN = 2000
NB = NU // BN


def _ln(x, g, b, eps=1e-5):
    mu = jnp.mean(x, axis=-1, keepdims=True)
    var = jnp.mean((x - mu) ** 2, axis=-1, keepdims=True)
    return (x - mu) * lax.rsqrt(var + eps) * g + b


def _stage_a_body(xi_ref, g_ref, b_ref, o0_ref, o1_ref):
    x = xi_ref[...]
    y = jnp.maximum(_ln(x, g_ref[...], b_ref[...]), 0.0)
    for c in range(NF):
        o0_ref[c] = x[:, c * HF:(c + 1) * HF]
        o1_ref[c] = y[:, c * HF:(c + 1) * HF]


def _stage_a(xi0, g, b):
    return pl.pallas_call(
        _stage_a_body,
        grid=(NB,),
        in_specs=[
            pl.BlockSpec((BN, H), lambda i: (i, 0)),
            pl.BlockSpec((1, H), lambda i: (0, 0)),
            pl.BlockSpec((1, H), lambda i: (0, 0)),
        ],
        out_specs=[
            pl.BlockSpec((NF, BN, HF), lambda i: (0, i, 0)),
            pl.BlockSpec((NF, BN, HF), lambda i: (0, i, 0)),
        ],
        out_shape=[
            jax.ShapeDtypeStruct((NF, NUP, HF), jnp.float32),
            jax.ShapeDtypeStruct((NF, NUP, HF), jnp.float32),
        ],
    )(xi0, g.reshape(1, H), b.reshape(1, H))


EB = 128
EROWS = 2560
NBR = EROWS // 16
RB2 = 320


def _stage_a2_body(g_ref, h_ref, o_ref, p_ref):
    c = pl.program_id(0)
    o_ref[0] = g_ref[...] + c * NUP
    p_ref[0] = h_ref[...] + c * NUP


def _stage_a2(g2d, h2d):
    return pl.pallas_call(
        _stage_a2_body,
        grid=(NF, EROWS // RB2),
        in_specs=[pl.BlockSpec((RB2, EB), lambda c, r: (r, 0)),
                  pl.BlockSpec((RB2, EB), lambda c, r: (r, 0))],
        out_specs=[pl.BlockSpec((1, RB2, EB), lambda c, r: (c, r, 0)),
                   pl.BlockSpec((1, RB2, EB), lambda c, r: (c, r, 0))],
        out_shape=[jax.ShapeDtypeStruct((NF, EROWS, EB), jnp.int32),
                   jax.ShapeDtypeStruct((NF, EROWS, EB), jnp.int32)],
    )(g2d, h2d)


NSLOT = 5
RSTAGE = 40
NWIN = NBR // RSTAGE
NRND = RSTAGE // NSLOT
RPT = NUP // 16
ZCH = 128


def _sc_body(do_counts, *refs):
    if do_counts:
        (tblr, gfk4, grv4, sfk, srv,
         ma, mb, c1o, c2o,
         gst, sst, rows, acc, gsem, ssem, zsem) = refs
    else:
        (tblr, gfk4, grv4, sfk, srv,
         ma, mb,
         gst, sst, rows, acc, gsem, ssem, zsem) = refs
    core = lax.axis_index("c")
    sub = lax.axis_index("s")
    row0 = sub * RPT
    brow0 = sub * NBR

    def fill_slot0(val):
        def body(i, _):
            for k in range(HF // 16):
                rows[0, i, pl.ds(k * 16, 16)] = jnp.full((16,), val,
                                                         jnp.float32)
            return 0
        lax.fori_loop(0, EB, body, 0)

    def zero_acc():
        fill_slot0(0.0)
        zsrc = rows.at[0]
        copies = [pltpu.async_copy(
            zsrc.at[pl.ds(0, ZCH)] if z < 24 else zsrc.at[pl.ds(0, 56)],
            acc.at[pl.ds(row0 + z * ZCH, ZCH if z < 24 else 56)],
            zsem) for z in range(25)]
        for c in copies:
            c.wait()

    def edge_pass(tbl, g4, s2d, chunk):
        def window(w, _):
            wrow = brow0 + w * RSTAGE
            pltpu.sync_copy(s2d.at[pl.ds(wrow, RSTAGE)], sst)
            if tbl is not None:
                pltpu.sync_copy(g4.at[chunk, pl.ds(wrow, RSTAGE)], gst)

                def g_start(k, lr):
                    pltpu.async_copy(tbl.at[gst.at[lr]], rows.at[k],
                                     gsem.at[k])

                def g_wait(k, lr):
                    pltpu.make_async_copy(tbl.at[gst.at[lr]], rows.at[k],
                                          gsem.at[k]).wait()

                def s_start(k, lr):
                    pltpu.async_copy(rows.at[k], acc.at[sst.at[lr]],
                                     ssem.at[k], add=True)

                def s_wait(k, lr):
                    pltpu.make_async_copy(rows.at[k], acc.at[sst.at[lr]],
                                          ssem.at[k]).wait()

                for k in range(NSLOT):
                    g_start(k, k)

                def rnd(r, _):
                    for k in range(NSLOT):
                        lr = r * NSLOT + k
                        g_wait(k, lr)
                        s_start(k, lr)
                    for k in range(NSLOT):
                        lr = r * NSLOT + k
                        s_wait(k, lr)

                        @pl.when(r < NRND - 1)
                        def _():
                            g_start(k, lr + NSLOT)
                    return 0
                lax.fori_loop(0, NRND, rnd, 0)
            else:
                def rnd(r, _):
                    copies = [pltpu.async_copy(
                        rows.at[0], acc.at[sst.at[r * NSLOT + k]],
                        ssem.at[k], add=True) for k in range(NSLOT)]
                    for c in copies:
                        c.wait()
                    return 0
                lax.fori_loop(0, NRND, rnd, 0)
            return 0
        lax.fori_loop(0, NWIN, window, 0)

    def writeback(out, toff):
        pltpu.sync_copy(acc.at[pl.ds(row0, RPT)],
                        out.at[pl.ds(toff + row0, RPT)])

    if do_counts:
        zero_acc()
        fill_slot0(1.0)
        plsc.subcore_barrier()

        @pl.when(core == 0)
        def _():
            edge_pass(None, None, sfk, 0)

        @pl.when(core == 1)
        def _():
            edge_pass(None, None, srv, 0)
        plsc.subcore_barrier()

        @pl.when(core == 0)
        def _():
            writeback(c1o, 0)

        @pl.when(core == 1)
        def _():
            writeback(c2o, 0)
        plsc.subcore_barrier()

    for g4, s2d, out in ((gfk4, sfk, ma), (grv4, srv, mb)):
        def chunk_body(j, _, g4=g4, s2d=s2d, out=out):
            chunk = core * 2 + j
            zero_acc()
            plsc.subcore_barrier()
            edge_pass(tblr, g4, s2d, chunk)
            plsc.subcore_barrier()
            writeback(out, chunk * NUP)
            plsc.subcore_barrier()
            return 0
        lax.fori_loop(0, 2, chunk_body, 0)


def _seg_means_sc(tbl, gfk4, grv4, sfk, srv, do_counts):
    m_ty = jax.ShapeDtypeStruct((NF * NUP, HF), jnp.float32)
    c_ty = jax.ShapeDtypeStruct((NUP, HF), jnp.float32)
    f = pl.kernel(
        functools.partial(_sc_body, do_counts),
        out_type=[m_ty, m_ty] + ([c_ty, c_ty] if do_counts else []),
        mesh=plsc.VectorSubcoreMesh(core_axis_name="c", subcore_axis_name="s"),
        compiler_params=pltpu.CompilerParams(use_tc_tiling_on_sc=False),
        scratch_types=[
            pltpu.VMEM((RSTAGE, EB), jnp.int32),
            pltpu.VMEM((RSTAGE, EB), jnp.int32),
            pltpu.VMEM((NSLOT, EB, HF), jnp.float32),
            pltpu.VMEM_SHARED((NUP, HF), jnp.float32),
            pltpu.SemaphoreType.DMA((NSLOT,)),
            pltpu.SemaphoreType.DMA((NSLOT,)),
            pltpu.SemaphoreType.DMA,
        ],
    )
    outs = f(tbl.reshape(NF * NUP, HF), gfk4, grv4, sfk, srv)
    rs = lambda m: m.reshape(NF, NUP, HF)
    if do_counts:
        ma, mb, c1, c2 = outs
        return rs(ma), rs(mb), c1, c2
    ma, mb = outs
    return rs(ma), rs(mb)


def _pad2d(x, fill):
    x = x.reshape(E // EB, EB)
    pad = jnp.full((EROWS - E // EB, EB), fill, jnp.int32)
    return jnp.concatenate([x, pad], axis=0)


def _layer_body(z_is_input, *refs):
    if z_is_input:
        (xu_ref, m1_ref, m2_ref, c1_ref, c2_ref, z_ref,
         w1_ref, b1_ref, w2_ref, b2_ref, wu_ref, bu_ref,
         pg_ref, pb_ref, wl_ref, bl_ref, wr_ref, ng_ref, nb_ref,
         o_ref) = refs
    else:
        (xu_ref, m1_ref, m2_ref, c1_ref, c2_ref,
         w1_ref, b1_ref, w2_ref, b2_ref, wu_ref, bu_ref,
         pg_ref, pb_ref, wl_ref, bl_ref, wr_ref, ng_ref, nb_ref,
         o_ref) = refs
        z_ref = None
    xu = xu_ref[...]
    c1 = c1_ref[:, 0:1]
    c2 = c2_ref[:, 0:1]
    r1 = 1.0 / jnp.maximum(c1, 1.0)
    r2 = 1.0 / jnp.maximum(c2, 1.0)
    has1 = (c1 > 0.0).astype(jnp.float32)

    if z_ref is None:
        z = jnp.maximum(
            jnp.dot(xu, w1_ref[...], preferred_element_type=jnp.float32)
            + b1_ref[...], 0.0)
        z = (jnp.dot(z, w2_ref[...], preferred_element_type=jnp.float32)
             + b2_ref[...])
    else:
        z = z_ref[...]

    upd = bu_ref[...]
    sage = bl_ref[...] + jnp.dot(xu, wr_ref[...],
                                 preferred_element_type=jnp.float32)
    for c in range(NF):
        sl = slice(c * HF, (c + 1) * HF)
        agg_c = m1_ref[c] * r1 - z[:, sl] * has1
        upd = upd + jnp.dot(agg_c, wu_ref[sl, :],
                            preferred_element_type=jnp.float32)
        sage = sage + jnp.dot(m2_ref[c] * r2, wl_ref[sl, :],
                              preferred_element_type=jnp.float32)

    prmp = _ln(xu + upd, pg_ref[...], pb_ref[...])
    merged = (prmp + sage) * 0.5
    o_ref[...] = jnp.maximum(_ln(merged, ng_ref[...], nb_ref[...]), 0.0)


def _layer_dense(xu, m1, m2, cnt1, cnt2, p, z=None):
    vec = lambda v: v.reshape(1, H)
    full = lambda shp: pl.BlockSpec(shp, lambda i: (0, 0))
    zspec = [pl.BlockSpec((BN, H), lambda i: (i, 0))] if z is not None else []
    zarg = [z] if z is not None else []
    return pl.pallas_call(
        functools.partial(_layer_body, z is not None),
        grid=(NB,),
        in_specs=[
            pl.BlockSpec((BN, H), lambda i: (i, 0)),
            pl.BlockSpec((NF, BN, HF), lambda i: (0, i, 0)),
            pl.BlockSpec((NF, BN, HF), lambda i: (0, i, 0)),
            pl.BlockSpec((BN, HF), lambda i: (i, 0)),
            pl.BlockSpec((BN, HF), lambda i: (i, 0)),
        ] + zspec + [
            full((H, H)), full((1, H)), full((H, H)), full((1, H)),
            full((H, H)), full((1, H)), full((1, H)), full((1, H)),
            full((H, H)), full((1, H)), full((H, H)), full((1, H)),
            full((1, H)),
        ],
        out_specs=pl.BlockSpec((BN, H), lambda i: (i, 0)),
        out_shape=jax.ShapeDtypeStruct((NU, H), jnp.float32),
    )(xu, m1, m2, cnt1, cnt2, *zarg,
      p['W1'], vec(p['b1']), p['W2'], vec(p['b2']),
      p['Wu'], vec(p['bu']), vec(p['pg']), vec(p['pb']),
      p['Wl'], vec(p['bl']), p['Wr'], vec(p['nug']), vec(p['nub']))


def _zmlp_body(xu_ref, w1_ref, b1_ref, w2_ref, b2_ref, o_ref):
    z = jnp.maximum(
        jnp.dot(xu_ref[...], w1_ref[...], preferred_element_type=jnp.float32)
        + b1_ref[...], 0.0)
    o_ref[...] = (jnp.dot(z, w2_ref[...], preferred_element_type=jnp.float32)
                  + b2_ref[...])


def _zmlp(xu, p):
    vec = lambda v: v.reshape(1, H)
    full = lambda shp: pl.BlockSpec(shp, lambda i: (0, 0))
    return pl.pallas_call(
        _zmlp_body,
        grid=(NB,),
        in_specs=[
            pl.BlockSpec((BN, H), lambda i: (i, 0)),
            full((H, H)), full((1, H)), full((H, H)), full((1, H)),
        ],
        out_specs=pl.BlockSpec((BN, H), lambda i: (i, 0)),
        out_shape=jax.ShapeDtypeStruct((NU, H), jnp.float32),
    )(xu, p['W1'], vec(p['b1']), p['W2'], vec(p['b2']))


def _head_body(h_ref, w1_ref, b1_ref, w2_ref, b2_ref, o_ref):
    h = jnp.maximum(
        jnp.dot(h_ref[...], w1_ref[...], preferred_element_type=jnp.float32)
        + b1_ref[...], 0.0)
    o_ref[...] = (jnp.dot(h, w2_ref[...], preferred_element_type=jnp.float32)
                  + b2_ref[...])


def _head(hrows, hW1, hb1, hW2, hb2):
    BH = 1024
    out = pl.pallas_call(
        _head_body,
        grid=(B // BH,),
        in_specs=[
            pl.BlockSpec((BH, H), lambda i: (i, 0)),
            pl.BlockSpec((H, H // 2), lambda i: (0, 0)),
            pl.BlockSpec((1, H // 2), lambda i: (0, 0)),
            pl.BlockSpec((H // 2, 1), lambda i: (0, 0)),
            pl.BlockSpec((1, 1), lambda i: (0, 0)),
        ],
        out_specs=pl.BlockSpec((BH, 1), lambda i: (i, 0)),
        out_shape=jax.ShapeDtypeStruct((B, 1), jnp.float32),
    )(hrows, hW1, hb1.reshape(1, H // 2), hW2, hb2.reshape(1, 1))
    return out.reshape(B)


def kernel(params, edge_fk, edge_rev, target_ids):
    p0 = params['layer0']
    xi0 = params['emb_item']
    xu = params['emb_user']

    xi0c, xi1c = _stage_a(xi0, p0['nig'], p0['nib'])
    gfk4, grv4 = _stage_a2(_pad2d(edge_fk[1], 0), _pad2d(edge_rev[0], 0))
    sfk = _pad2d(edge_fk[0], NU)
    srv = _pad2d(edge_rev[1], NU)
    z0 = _zmlp(xu, p0)
    m10, m20, c1, c2 = _seg_means_sc(xi0c, gfk4, grv4, sfk, srv, True)
    m11, m21 = _seg_means_sc(xi1c, gfk4, grv4, sfk, srv, False)

    xu = _layer_dense(xu, m10, m20, c1, c2, params['layer0'], z=z0)
    xu = _layer_dense(xu, m11, m21, c1, c2, params['layer1'])

    hrows = xu[target_ids]
    return _head(hrows, params['hW1'], params['hb1'], params['hW2'], params['hb2'])


# speedup vs baseline: 1.0119x; 1.0040x over previous
import functools

import jax
import jax.numpy as jnp
from jax import lax
from jax.experimental import pallas as pl
from jax.experimental.pallas import tpu as pltpu
from jax.experimental.pallas import tpu_sc as plsc

H = 128
NU = 50000
NI = 50000
E = 320000
NL = 2
B = 8192
NUP = 50048
NF = 4
HF = H // NF
BN = 2000
NB = NU // BN


def _ln(x, g, b, eps=1e-5):
    mu = jnp.mean(x, axis=-1, keepdims=True)
    var = jnp.mean((x - mu) ** 2, axis=-1, keepdims=True)
    return (x - mu) * lax.rsqrt(var + eps) * g + b


def _stage_a_body(xi_ref, g_ref, b_ref, o0_ref, o1_ref):
    x = xi_ref[...]
    y = jnp.maximum(_ln(x, g_ref[...], b_ref[...]), 0.0)
    for c in range(NF):
        o0_ref[c] = x[:, c * HF:(c + 1) * HF]
        o1_ref[c] = y[:, c * HF:(c + 1) * HF]


def _stage_a(xi0, g, b):
    return pl.pallas_call(
        _stage_a_body,
        grid=(NB,),
        in_specs=[
            pl.BlockSpec((BN, H), lambda i: (i, 0)),
            pl.BlockSpec((1, H), lambda i: (0, 0)),
            pl.BlockSpec((1, H), lambda i: (0, 0)),
        ],
        out_specs=[
            pl.BlockSpec((NF, BN, HF), lambda i: (0, i, 0)),
            pl.BlockSpec((NF, BN, HF), lambda i: (0, i, 0)),
        ],
        out_shape=[
            jax.ShapeDtypeStruct((NF, NUP, HF), jnp.float32),
            jax.ShapeDtypeStruct((NF, NUP, HF), jnp.float32),
        ],
    )(xi0, g.reshape(1, H), b.reshape(1, H))


EB = 128
EROWS = 2560
NBR = EROWS // 16
RB2 = 320


def _stage_a2_body(g_ref, h_ref, o_ref, p_ref):
    c = pl.program_id(0)
    o_ref[0] = g_ref[...] + c * NUP
    p_ref[0] = h_ref[...] + c * NUP


def _stage_a2(g2d, h2d):
    return pl.pallas_call(
        _stage_a2_body,
        grid=(NF, EROWS // RB2),
        in_specs=[pl.BlockSpec((RB2, EB), lambda c, r: (r, 0)),
                  pl.BlockSpec((RB2, EB), lambda c, r: (r, 0))],
        out_specs=[pl.BlockSpec((1, RB2, EB), lambda c, r: (c, r, 0)),
                   pl.BlockSpec((1, RB2, EB), lambda c, r: (c, r, 0))],
        out_shape=[jax.ShapeDtypeStruct((NF, EROWS, EB), jnp.int32),
                   jax.ShapeDtypeStruct((NF, EROWS, EB), jnp.int32)],
    )(g2d, h2d)


NSLOT = 5
RSTAGE = 40
NWIN = NBR // RSTAGE
NRND = RSTAGE // NSLOT
RPT = NUP // 16
ZCH = 128


def _sc_body(do_counts, *refs):
    if do_counts:
        (tblr, gfk4, grv4, sfk, srv,
         ma, mb, c1o, c2o,
         gst, sst, rows, acc, gsem, ssem, zsem) = refs
    else:
        (tblr, gfk4, grv4, sfk, srv,
         ma, mb,
         gst, sst, rows, acc, gsem, ssem, zsem) = refs
    core = lax.axis_index("c")
    sub = lax.axis_index("s")
    row0 = sub * RPT
    brow0 = sub * NBR

    def fill_slot0(val):
        def body(i, _):
            for k in range(HF // 16):
                rows[0, i, pl.ds(k * 16, 16)] = jnp.full((16,), val,
                                                         jnp.float32)
            return 0
        lax.fori_loop(0, EB, body, 0)

    def zero_acc():
        fill_slot0(0.0)
        zsrc = rows.at[0]
        copies = [pltpu.async_copy(
            zsrc.at[pl.ds(0, ZCH)] if z < 24 else zsrc.at[pl.ds(0, 56)],
            acc.at[pl.ds(row0 + z * ZCH, ZCH if z < 24 else 56)],
            zsem) for z in range(25)]
        for c in copies:
            c.wait()

    def edge_pass(tbl, g4, s2d, chunk):
        def window(w, _):
            wrow = brow0 + w * RSTAGE
            pltpu.sync_copy(s2d.at[pl.ds(wrow, RSTAGE)], sst)
            if tbl is not None:
                pltpu.sync_copy(g4.at[chunk, pl.ds(wrow, RSTAGE)], gst)

                def g_start(k, lr):
                    pltpu.async_copy(tbl.at[gst.at[lr]], rows.at[k],
                                     gsem.at[k])

                def g_wait(k, lr):
                    pltpu.make_async_copy(tbl.at[gst.at[lr]], rows.at[k],
                                          gsem.at[k]).wait()

                def s_start(k, lr):
                    pltpu.async_copy(rows.at[k], acc.at[sst.at[lr]],
                                     ssem.at[k], add=True)

                def s_wait(k, lr):
                    pltpu.make_async_copy(rows.at[k], acc.at[sst.at[lr]],
                                          ssem.at[k]).wait()

                for k in range(NSLOT):
                    g_start(k, k)

                def rnd(r, _):
                    for k in range(NSLOT):
                        lr = r * NSLOT + k
                        g_wait(k, lr)
                        s_start(k, lr)
                    for k in range(NSLOT):
                        lr = r * NSLOT + k
                        s_wait(k, lr)

                        @pl.when(r < NRND - 1)
                        def _():
                            g_start(k, lr + NSLOT)
                    return 0
                lax.fori_loop(0, NRND, rnd, 0)
            else:
                def rnd(r, _):
                    copies = [pltpu.async_copy(
                        rows.at[0], acc.at[sst.at[r * NSLOT + k]],
                        ssem.at[k], add=True) for k in range(NSLOT)]
                    for c in copies:
                        c.wait()
                    return 0
                lax.fori_loop(0, NRND, rnd, 0)
            return 0
        lax.fori_loop(0, NWIN, window, 0)

    def writeback(out, toff):
        pltpu.sync_copy(acc.at[pl.ds(row0, RPT)],
                        out.at[pl.ds(toff + row0, RPT)])

    if do_counts:
        zero_acc()
        fill_slot0(1.0)
        plsc.subcore_barrier()

        @pl.when(core == 0)
        def _():
            edge_pass(None, None, sfk, 0)

        @pl.when(core == 1)
        def _():
            edge_pass(None, None, srv, 0)
        plsc.subcore_barrier()

        @pl.when(core == 0)
        def _():
            writeback(c1o, 0)

        @pl.when(core == 1)
        def _():
            writeback(c2o, 0)
        plsc.subcore_barrier()

    for g4, s2d, out in ((gfk4, sfk, ma), (grv4, srv, mb)):
        def chunk_body(j, _, g4=g4, s2d=s2d, out=out):
            chunk = core * 2 + j
            zero_acc()
            plsc.subcore_barrier()
            edge_pass(tblr, g4, s2d, chunk)
            plsc.subcore_barrier()
            writeback(out, chunk * NUP)
            plsc.subcore_barrier()
            return 0
        lax.fori_loop(0, 2, chunk_body, 0)


def _seg_means_sc(tbl, gfk4, grv4, sfk, srv, do_counts):
    m_ty = jax.ShapeDtypeStruct((NF * NUP, HF), jnp.float32)
    c_ty = jax.ShapeDtypeStruct((NUP, HF), jnp.float32)
    f = pl.kernel(
        functools.partial(_sc_body, do_counts),
        out_type=[m_ty, m_ty] + ([c_ty, c_ty] if do_counts else []),
        mesh=plsc.VectorSubcoreMesh(core_axis_name="c", subcore_axis_name="s"),
        compiler_params=pltpu.CompilerParams(use_tc_tiling_on_sc=False),
        scratch_types=[
            pltpu.VMEM((RSTAGE, EB), jnp.int32),
            pltpu.VMEM((RSTAGE, EB), jnp.int32),
            pltpu.VMEM((NSLOT, EB, HF), jnp.float32),
            pltpu.VMEM_SHARED((NUP, HF), jnp.float32),
            pltpu.SemaphoreType.DMA((NSLOT,)),
            pltpu.SemaphoreType.DMA((NSLOT,)),
            pltpu.SemaphoreType.DMA,
        ],
    )
    outs = f(tbl.reshape(NF * NUP, HF), gfk4, grv4, sfk, srv)
    rs = lambda m: m.reshape(NF, NUP, HF)
    if do_counts:
        ma, mb, c1, c2 = outs
        return rs(ma), rs(mb), c1, c2
    ma, mb = outs
    return rs(ma), rs(mb)


def _pad2d(x, fill):
    x = x.reshape(E // EB, EB)
    pad = jnp.full((EROWS - E // EB, EB), fill, jnp.int32)
    return jnp.concatenate([x, pad], axis=0)


def _layer_body(z_is_input, *refs):
    if z_is_input:
        (xu_ref, m1_ref, m2_ref, c1_ref, c2_ref, z_ref,
         w1_ref, b1_ref, w2_ref, b2_ref, wu_ref, bu_ref,
         pg_ref, pb_ref, wl_ref, bl_ref, wr_ref, ng_ref, nb_ref,
         o_ref) = refs
    else:
        (xu_ref, m1_ref, m2_ref, c1_ref, c2_ref,
         w1_ref, b1_ref, w2_ref, b2_ref, wu_ref, bu_ref,
         pg_ref, pb_ref, wl_ref, bl_ref, wr_ref, ng_ref, nb_ref,
         o_ref) = refs
        z_ref = None
    xu = xu_ref[...]
    c1 = c1_ref[:, 0:1]
    c2 = c2_ref[:, 0:1]
    r1 = 1.0 / jnp.maximum(c1, 1.0)
    r2 = 1.0 / jnp.maximum(c2, 1.0)
    has1 = (c1 > 0.0).astype(jnp.float32)

    if z_ref is None:
        z = jnp.maximum(
            jnp.dot(xu, w1_ref[...], preferred_element_type=jnp.float32)
            + b1_ref[...], 0.0)
        z = (jnp.dot(z, w2_ref[...], preferred_element_type=jnp.float32)
             + b2_ref[...])
    else:
        z = z_ref[...]

    upd = bu_ref[...]
    sage = bl_ref[...] + jnp.dot(xu, wr_ref[...],
                                 preferred_element_type=jnp.float32)
    for c in range(NF):
        sl = slice(c * HF, (c + 1) * HF)
        agg_c = m1_ref[c] * r1 - z[:, sl] * has1
        upd = upd + jnp.dot(agg_c, wu_ref[sl, :],
                            preferred_element_type=jnp.float32)
        sage = sage + jnp.dot(m2_ref[c] * r2, wl_ref[sl, :],
                              preferred_element_type=jnp.float32)

    prmp = _ln(xu + upd, pg_ref[...], pb_ref[...])
    merged = (prmp + sage) * 0.5
    o_ref[...] = jnp.maximum(_ln(merged, ng_ref[...], nb_ref[...]), 0.0)


def _layer_dense(xu, m1, m2, cnt1, cnt2, p, z=None):
    vec = lambda v: v.reshape(1, H)
    full = lambda shp: pl.BlockSpec(shp, lambda i: (0, 0))
    zspec = [pl.BlockSpec((BN, H), lambda i: (i, 0))] if z is not None else []
    zarg = [z] if z is not None else []
    return pl.pallas_call(
        functools.partial(_layer_body, z is not None),
        grid=(NB,),
        in_specs=[
            pl.BlockSpec((BN, H), lambda i: (i, 0)),
            pl.BlockSpec((NF, BN, HF), lambda i: (0, i, 0)),
            pl.BlockSpec((NF, BN, HF), lambda i: (0, i, 0)),
            pl.BlockSpec((BN, HF), lambda i: (i, 0)),
            pl.BlockSpec((BN, HF), lambda i: (i, 0)),
        ] + zspec + [
            full((H, H)), full((1, H)), full((H, H)), full((1, H)),
            full((H, H)), full((1, H)), full((1, H)), full((1, H)),
            full((H, H)), full((1, H)), full((H, H)), full((1, H)),
            full((1, H)),
        ],
        out_specs=pl.BlockSpec((BN, H), lambda i: (i, 0)),
        out_shape=jax.ShapeDtypeStruct((NU, H), jnp.float32),
    )(xu, m1, m2, cnt1, cnt2, *zarg,
      p['W1'], vec(p['b1']), p['W2'], vec(p['b2']),
      p['Wu'], vec(p['bu']), vec(p['pg']), vec(p['pb']),
      p['Wl'], vec(p['bl']), p['Wr'], vec(p['nug']), vec(p['nub']))


def _zmlp_body(xu_ref, w1_ref, b1_ref, w2_ref, b2_ref, o_ref):
    z = jnp.maximum(
        jnp.dot(xu_ref[...], w1_ref[...], preferred_element_type=jnp.float32)
        + b1_ref[...], 0.0)
    o_ref[...] = (jnp.dot(z, w2_ref[...], preferred_element_type=jnp.float32)
                  + b2_ref[...])


def _zmlp(xu, p):
    vec = lambda v: v.reshape(1, H)
    full = lambda shp: pl.BlockSpec(shp, lambda i: (0, 0))
    return pl.pallas_call(
        _zmlp_body,
        grid=(NB,),
        in_specs=[
            pl.BlockSpec((BN, H), lambda i: (i, 0)),
            full((H, H)), full((1, H)), full((H, H)), full((1, H)),
        ],
        out_specs=pl.BlockSpec((BN, H), lambda i: (i, 0)),
        out_shape=jax.ShapeDtypeStruct((NU, H), jnp.float32),
    )(xu, p['W1'], vec(p['b1']), p['W2'], vec(p['b2']))


def _head_body(h_ref, w1_ref, b1_ref, w2_ref, b2_ref, o_ref):
    h = jnp.maximum(
        jnp.dot(h_ref[...], w1_ref[...], preferred_element_type=jnp.float32)
        + b1_ref[...], 0.0)
    o_ref[...] = (jnp.dot(h, w2_ref[...], preferred_element_type=jnp.float32)
                  + b2_ref[...])


def _head(hrows, hW1, hb1, hW2, hb2):
    BH = 1024
    out = pl.pallas_call(
        _head_body,
        grid=(B // BH,),
        in_specs=[
            pl.BlockSpec((BH, H), lambda i: (i, 0)),
            pl.BlockSpec((H, H // 2), lambda i: (0, 0)),
            pl.BlockSpec((1, H // 2), lambda i: (0, 0)),
            pl.BlockSpec((H // 2, 1), lambda i: (0, 0)),
            pl.BlockSpec((1, 1), lambda i: (0, 0)),
        ],
        out_specs=pl.BlockSpec((BH, 1), lambda i: (i, 0)),
        out_shape=jax.ShapeDtypeStruct((B, 1), jnp.float32),
    )(hrows, hW1, hb1.reshape(1, H // 2), hW2, hb2.reshape(1, 1))
    return out.reshape(B)


def kernel(params, edge_fk, edge_rev, target_ids):
    p0 = params['layer0']
    xi0 = params['emb_item']
    xu = params['emb_user']

    xi0c, xi1c = _stage_a(xi0, p0['nig'], p0['nib'])
    gfk4, grv4 = _stage_a2(_pad2d(edge_fk[1], 0), _pad2d(edge_rev[0], 0))
    sfk = _pad2d(edge_fk[0], NU)
    srv = _pad2d(edge_rev[1], NU)
    m10, m20, c1, c2 = _seg_means_sc(xi0c, gfk4, grv4, sfk, srv, True)
    m11, m21 = _seg_means_sc(xi1c, gfk4, grv4, sfk, srv, False)

    xu = _layer_dense(xu, m10, m20, c1, c2, params['layer0'])
    xu = _layer_dense(xu, m11, m21, c1, c2, params['layer1'])

    hrows = xu[target_ids]
    return _head(hrows, params['hW1'], params['hb1'], params['hW2'], params['hb2'])


# speedup vs baseline: 1.0127x; 1.0008x over previous
import functools

import jax
import jax.numpy as jnp
from jax import lax
from jax.experimental import pallas as pl
from jax.experimental.pallas import tpu as pltpu
from jax.experimental.pallas import tpu_sc as plsc

H = 128
NU = 50000
NI = 50000
E = 320000
NL = 2
B = 8192
NUP = 50048
NF = 4
HF = H // NF
BN = 2000
NB = NU // BN


def _ln(x, g, b, eps=1e-5):
    mu = jnp.mean(x, axis=-1, keepdims=True)
    var = jnp.mean((x - mu) ** 2, axis=-1, keepdims=True)
    return (x - mu) * lax.rsqrt(var + eps) * g + b


def _stage_a_body(xi_ref, g_ref, b_ref, o0_ref, o1_ref):
    x = xi_ref[...]
    y = jnp.maximum(_ln(x, g_ref[...], b_ref[...]), 0.0)
    for c in range(NF):
        o0_ref[c] = x[:, c * HF:(c + 1) * HF]
        o1_ref[c] = y[:, c * HF:(c + 1) * HF]


def _stage_a(xi0, g, b):
    return pl.pallas_call(
        _stage_a_body,
        grid=(NB,),
        in_specs=[
            pl.BlockSpec((BN, H), lambda i: (i, 0)),
            pl.BlockSpec((1, H), lambda i: (0, 0)),
            pl.BlockSpec((1, H), lambda i: (0, 0)),
        ],
        out_specs=[
            pl.BlockSpec((NF, BN, HF), lambda i: (0, i, 0)),
            pl.BlockSpec((NF, BN, HF), lambda i: (0, i, 0)),
        ],
        out_shape=[
            jax.ShapeDtypeStruct((NF, NUP, HF), jnp.float32),
            jax.ShapeDtypeStruct((NF, NUP, HF), jnp.float32),
        ],
    )(xi0, g.reshape(1, H), b.reshape(1, H))


EB = 128
EROWS = 2560
NBR = EROWS // 16
RB2 = 320


def _stage_a2_body(g_ref, h_ref, o_ref, p_ref):
    c = pl.program_id(0)
    o_ref[0] = g_ref[...] + c * NUP
    p_ref[0] = h_ref[...] + c * NUP


def _stage_a2(g2d, h2d):
    return pl.pallas_call(
        _stage_a2_body,
        grid=(NF, EROWS // RB2),
        in_specs=[pl.BlockSpec((RB2, EB), lambda c, r: (r, 0)),
                  pl.BlockSpec((RB2, EB), lambda c, r: (r, 0))],
        out_specs=[pl.BlockSpec((1, RB2, EB), lambda c, r: (c, r, 0)),
                   pl.BlockSpec((1, RB2, EB), lambda c, r: (c, r, 0))],
        out_shape=[jax.ShapeDtypeStruct((NF, EROWS, EB), jnp.int32),
                   jax.ShapeDtypeStruct((NF, EROWS, EB), jnp.int32)],
    )(g2d, h2d)


NSLOT = 5
RSTAGE = 40
NWIN = NBR // RSTAGE
NRND = RSTAGE // NSLOT
RPT = NUP // 16
ZCH = 128


def _sc_body(do_counts, *refs):
    if do_counts:
        (tblr, gfk4, grv4, sfk, srv,
         ma, mb, c1o, c2o,
         gst, sst, rows, acc, gsem, ssem, zsem) = refs
    else:
        (tblr, gfk4, grv4, sfk, srv,
         ma, mb,
         gst, sst, rows, acc, gsem, ssem, zsem) = refs
    core = lax.axis_index("c")
    sub = lax.axis_index("s")
    row0 = sub * RPT
    brow0 = sub * NBR

    def fill_slot0(val):
        def body(i, _):
            for k in range(HF // 16):
                rows[0, i, pl.ds(k * 16, 16)] = jnp.full((16,), val,
                                                         jnp.float32)
            return 0
        lax.fori_loop(0, EB, body, 0)

    def zero_acc():
        fill_slot0(0.0)
        zsrc = rows.at[0]
        copies = [pltpu.async_copy(
            zsrc.at[pl.ds(0, ZCH)] if z < 24 else zsrc.at[pl.ds(0, 56)],
            acc.at[pl.ds(row0 + z * ZCH, ZCH if z < 24 else 56)],
            zsem) for z in range(25)]
        for c in copies:
            c.wait()

    def edge_pass(tbl, g4, s2d, chunk):
        def window(w, _):
            wrow = brow0 + w * RSTAGE
            pltpu.sync_copy(s2d.at[pl.ds(wrow, RSTAGE)], sst)
            if tbl is not None:
                pltpu.sync_copy(g4.at[chunk, pl.ds(wrow, RSTAGE)], gst)

                def g_start(k, lr):
                    pltpu.async_copy(tbl.at[gst.at[lr]], rows.at[k],
                                     gsem.at[k])

                def g_wait(k, lr):
                    pltpu.make_async_copy(tbl.at[gst.at[lr]], rows.at[k],
                                          gsem.at[k]).wait()

                def s_start(k, lr):
                    pltpu.async_copy(rows.at[k], acc.at[sst.at[lr]],
                                     ssem.at[k], add=True)

                def s_wait(k, lr):
                    pltpu.make_async_copy(rows.at[k], acc.at[sst.at[lr]],
                                          ssem.at[k]).wait()

                for k in range(NSLOT):
                    g_start(k, k)

                def rnd(r, _):
                    for k in range(NSLOT):
                        lr = r * NSLOT + k
                        g_wait(k, lr)
                        s_start(k, lr)
                    for k in range(NSLOT):
                        lr = r * NSLOT + k
                        s_wait(k, lr)

                        @pl.when(r < NRND - 1)
                        def _():
                            g_start(k, lr + NSLOT)
                    return 0
                lax.fori_loop(0, NRND, rnd, 0)
            else:
                def rnd(r, _):
                    copies = [pltpu.async_copy(
                        rows.at[0], acc.at[sst.at[r * NSLOT + k]],
                        ssem.at[k], add=True) for k in range(NSLOT)]
                    for c in copies:
                        c.wait()
                    return 0
                lax.fori_loop(0, NRND, rnd, 0)
            return 0
        lax.fori_loop(0, NWIN, window, 0)

    def writeback(out, toff):
        pltpu.sync_copy(acc.at[pl.ds(row0, RPT)],
                        out.at[pl.ds(toff + row0, RPT)])

    if do_counts:
        zero_acc()
        fill_slot0(1.0)
        plsc.subcore_barrier()

        @pl.when(core == 0)
        def _():
            edge_pass(None, None, sfk, 0)

        @pl.when(core == 1)
        def _():
            edge_pass(None, None, srv, 0)
        plsc.subcore_barrier()

        @pl.when(core == 0)
        def _():
            writeback(c1o, 0)

        @pl.when(core == 1)
        def _():
            writeback(c2o, 0)
        plsc.subcore_barrier()

    for g4, s2d, out in ((gfk4, sfk, ma), (grv4, srv, mb)):
        def chunk_body(j, _, g4=g4, s2d=s2d, out=out):
            chunk = core * 2 + j
            zero_acc()
            plsc.subcore_barrier()
            edge_pass(tblr, g4, s2d, chunk)
            plsc.subcore_barrier()
            writeback(out, chunk * NUP)
            plsc.subcore_barrier()
            return 0
        lax.fori_loop(0, 2, chunk_body, 0)


def _seg_means_sc(tbl, gfk4, grv4, sfk, srv, do_counts):
    m_ty = jax.ShapeDtypeStruct((NF * NUP, HF), jnp.float32)
    c_ty = jax.ShapeDtypeStruct((NUP, HF), jnp.float32)
    f = pl.kernel(
        functools.partial(_sc_body, do_counts),
        out_type=[m_ty, m_ty] + ([c_ty, c_ty] if do_counts else []),
        mesh=plsc.VectorSubcoreMesh(core_axis_name="c", subcore_axis_name="s"),
        compiler_params=pltpu.CompilerParams(use_tc_tiling_on_sc=False),
        scratch_types=[
            pltpu.VMEM((RSTAGE, EB), jnp.int32),
            pltpu.VMEM((RSTAGE, EB), jnp.int32),
            pltpu.VMEM((NSLOT, EB, HF), jnp.float32),
            pltpu.VMEM_SHARED((NUP, HF), jnp.float32),
            pltpu.SemaphoreType.DMA((NSLOT,)),
            pltpu.SemaphoreType.DMA((NSLOT,)),
            pltpu.SemaphoreType.DMA,
        ],
    )
    outs = f(tbl.reshape(NF * NUP, HF), gfk4, grv4, sfk, srv)
    rs = lambda m: m.reshape(NF, NUP, HF)
    if do_counts:
        ma, mb, c1, c2 = outs
        return rs(ma), rs(mb), c1, c2
    ma, mb = outs
    return rs(ma), rs(mb)


def _pad2d(x, fill):
    x = x.reshape(E // EB, EB)
    pad = jnp.full((EROWS - E // EB, EB), fill, jnp.int32)
    return jnp.concatenate([x, pad], axis=0)


def _layer_body(xu_ref, m1_ref, m2_ref, c1_ref, c2_ref,
                w1_ref, b1_ref, w2_ref, b2_ref, wu_ref, bu_ref,
                pg_ref, pb_ref, wl_ref, bl_ref, wr_ref, ng_ref, nb_ref,
                o_ref):
    xu = xu_ref[...]
    c1 = c1_ref[:, 0:1]
    c2 = c2_ref[:, 0:1]
    r1 = 1.0 / jnp.maximum(c1, 1.0)
    r2 = 1.0 / jnp.maximum(c2, 1.0)
    has1 = (c1 > 0.0).astype(jnp.float32)

    z = jnp.maximum(
        jnp.dot(xu, w1_ref[...], preferred_element_type=jnp.float32)
        + b1_ref[...], 0.0)
    z = (jnp.dot(z, w2_ref[...], preferred_element_type=jnp.float32)
         + b2_ref[...])

    upd = bu_ref[...]
    sage = bl_ref[...] + jnp.dot(xu, wr_ref[...],
                                 preferred_element_type=jnp.float32)
    for c in range(NF):
        sl = slice(c * HF, (c + 1) * HF)
        agg_c = m1_ref[c] * r1 - z[:, sl] * has1
        upd = upd + jnp.dot(agg_c, wu_ref[sl, :],
                            preferred_element_type=jnp.float32)
        sage = sage + jnp.dot(m2_ref[c] * r2, wl_ref[sl, :],
                              preferred_element_type=jnp.float32)

    prmp = _ln(xu + upd, pg_ref[...], pb_ref[...])
    merged = (prmp + sage) * 0.5
    o_ref[...] = jnp.maximum(_ln(merged, ng_ref[...], nb_ref[...]), 0.0)


def _layer_dense(xu, m1, m2, cnt1, cnt2, p):
    vec = lambda v: v.reshape(1, H)
    full = lambda shp: pl.BlockSpec(shp, lambda i: (0, 0))
    return pl.pallas_call(
        _layer_body,
        grid=(NB,),
        in_specs=[
            pl.BlockSpec((BN, H), lambda i: (i, 0)),
            pl.BlockSpec((NF, BN, HF), lambda i: (0, i, 0)),
            pl.BlockSpec((NF, BN, HF), lambda i: (0, i, 0)),
            pl.BlockSpec((BN, HF), lambda i: (i, 0)),
            pl.BlockSpec((BN, HF), lambda i: (i, 0)),
            full((H, H)), full((1, H)), full((H, H)), full((1, H)),
            full((H, H)), full((1, H)), full((1, H)), full((1, H)),
            full((H, H)), full((1, H)), full((H, H)), full((1, H)),
            full((1, H)),
        ],
        out_specs=pl.BlockSpec((BN, H), lambda i: (i, 0)),
        out_shape=jax.ShapeDtypeStruct((NU, H), jnp.float32),
    )(xu, m1, m2, cnt1, cnt2,
      p['W1'], vec(p['b1']), p['W2'], vec(p['b2']),
      p['Wu'], vec(p['bu']), vec(p['pg']), vec(p['pb']),
      p['Wl'], vec(p['bl']), p['Wr'], vec(p['nug']), vec(p['nub']))


def _head_body(h_ref, w1_ref, b1_ref, w2_ref, b2_ref, o_ref):
    h = jnp.maximum(
        jnp.dot(h_ref[...], w1_ref[...], preferred_element_type=jnp.float32)
        + b1_ref[...], 0.0)
    o_ref[...] = (jnp.dot(h, w2_ref[...], preferred_element_type=jnp.float32)
                  + b2_ref[...])


def _head(hrows, hW1, hb1, hW2, hb2):
    BH = 1024
    out = pl.pallas_call(
        _head_body,
        grid=(B // BH,),
        in_specs=[
            pl.BlockSpec((BH, H), lambda i: (i, 0)),
            pl.BlockSpec((H, H // 2), lambda i: (0, 0)),
            pl.BlockSpec((1, H // 2), lambda i: (0, 0)),
            pl.BlockSpec((H // 2, 1), lambda i: (0, 0)),
            pl.BlockSpec((1, 1), lambda i: (0, 0)),
        ],
        out_specs=pl.BlockSpec((BH, 1), lambda i: (i, 0)),
        out_shape=jax.ShapeDtypeStruct((B, 1), jnp.float32),
    )(hrows, hW1, hb1.reshape(1, H // 2), hW2, hb2.reshape(1, 1))
    return out.reshape(B)


def kernel(params, edge_fk, edge_rev, target_ids):
    p0 = params['layer0']
    xi0 = params['emb_item']
    xu = params['emb_user']

    xi0c, xi1c = _stage_a(xi0, p0['nig'], p0['nib'])
    gfk4, grv4 = _stage_a2(_pad2d(edge_fk[1], 0), _pad2d(edge_rev[0], 0))
    sfk = _pad2d(edge_fk[0], NU)
    srv = _pad2d(edge_rev[1], NU)
    m10, m20, c1, c2 = _seg_means_sc(xi0c, gfk4, grv4, sfk, srv, True)
    m11, m21 = _seg_means_sc(xi1c, gfk4, grv4, sfk, srv, False)

    xu = _layer_dense(xu, m10, m20, c1, c2, params['layer0'])
    xu = _layer_dense(xu, m11, m21, c1, c2, params['layer1'])

    hrows = xu[target_ids]
    return _head(hrows, params['hW1'], params['hb1'], params['hW2'], params['hb2'])
